# Initial kernel scaffold; baseline (speedup 1.0000x reference)
import jax
import jax.numpy as jnp
from jax.experimental import pallas as pl


def kernel(x, hyperedge_index, W1, b1, W2, b2, r):
    raise NotImplementedError("write your pallas kernel here")


# speedup vs baseline: 11.1379x; 11.1379x over previous
import functools
import jax
import jax.numpy as jnp
from jax.experimental import pallas as pl
from jax.experimental.pallas import tpu as pltpu

N = 10000
M = 10000
D = 128
H = 128
C = 16

_BLK = 1000


def _z():
    import jax.numpy as _jnp
    return _jnp.int32(0)


def _mm1_body(x_ref, w_ref, b_ref, r_ref, th_ref, s_ref):
    xb = x_ref[...]
    th_ref[...] = xb @ w_ref[...] + b_ref[...]
    s_ref[...] = xb @ r_ref[...]


def _fused_first(x, W1, b1, r):
    b2d = b1.reshape(1, H)
    r2d = r.reshape(D, 1)
    theta, s = pl.pallas_call(
        _mm1_body,
        grid=(N // _BLK,),
        in_specs=[
            pl.BlockSpec((_BLK, D), lambda i: (i, _z())),
            pl.BlockSpec((D, H), lambda i: (_z(), _z())),
            pl.BlockSpec((1, H), lambda i: (_z(), _z())),
            pl.BlockSpec((D, 1), lambda i: (_z(), _z())),
        ],
        out_specs=[
            pl.BlockSpec((_BLK, H), lambda i: (i, _z())),
            pl.BlockSpec((_BLK, 1), lambda i: (i, _z())),
        ],
        out_shape=[
            jax.ShapeDtypeStruct((N, H), jnp.float32),
            jax.ShapeDtypeStruct((N, 1), jnp.float32),
        ],
    )(x, W1, b2d, r2d)
    return theta, s.reshape(N)


def _mm2_body(h_ref, w_ref, b_ref, o_ref):
    o_ref[...] = jnp.maximum(h_ref[...], 0.0) @ w_ref[...] + b_ref[...]


def _fused_second(h1_pre, W2, b2):
    b2d = b2.reshape(1, C)
    return pl.pallas_call(
        _mm2_body,
        grid=(N // _BLK,),
        in_specs=[
            pl.BlockSpec((_BLK, H), lambda i: (i, _z())),
            pl.BlockSpec((H, C), lambda i: (_z(), _z())),
            pl.BlockSpec((1, C), lambda i: (_z(), _z())),
        ],
        out_specs=pl.BlockSpec((_BLK, C), lambda i: (i, _z())),
        out_shape=jax.ShapeDtypeStruct((N, C), jnp.float32),
    )(h1_pre, W2, b2d)


def kernel(x, hyperedge_index, W1, b1, W2, b2, r):
    node = hyperedge_index[0].astype(jnp.int32)
    he = hyperedge_index[1].astype(jnp.int32)

    theta_raw, s = _fused_first(x, W1, b1, r)

    sv = s[node]
    smax = jax.ops.segment_max(sv, he, num_segments=M)
    smin = jax.ops.segment_min(sv, he, num_segments=M)
    big = jnp.int32(N)
    cand_i = jnp.where(sv >= smax[he], node, big)
    i_e = jax.ops.segment_min(cand_i, he, num_segments=M)
    cand_j = jnp.where(sv <= smin[he], node, big)
    j_e = jax.ops.segment_min(cand_j, he, num_segments=M)
    i_e = jnp.clip(i_e, 0, N - 1)
    j_e = jnp.clip(j_e, 0, N - 1)

    deg = jnp.ones((N,), jnp.float32)
    deg = deg.at[i_e].add(1.0).at[j_e].add(1.0)
    dinv = jax.lax.rsqrt(deg)

    th1 = theta_raw * dinv[:, None]
    acc1 = th1
    acc1 = acc1.at[j_e].add(th1[i_e]).at[i_e].add(th1[j_e])
    h1_pre = acc1 * dinv[:, None]

    th2 = _fused_second(h1_pre, W2, b2) * dinv[:, None]
    acc2 = th2
    acc2 = acc2.at[j_e].add(th2[i_e]).at[i_e].add(th2[j_e])
    out = acc2 * dinv[:, None]
    return out


# speedup vs baseline: 11.5070x; 1.0331x over previous
import functools
import jax
import jax.numpy as jnp
from jax import lax
from jax.experimental import pallas as pl
from jax.experimental.pallas import tpu as pltpu
from jax.experimental.pallas import tpu_sc as plsc

N = 10000
M = 10000
EI = 320000
D = 128
H = 128
C = 16

NP = 10240
NT = 16
RPT = NP // NT
CH = 128
NCH = RPT // CH
DUMMY = 10100

_BLK = 1280


def _z():
    return jnp.int32(0)


def _mm1_body(x_ref, w_ref, b_ref, r_ref, th_ref, s_ref):
    xb = x_ref[...]
    th_ref[...] = xb @ w_ref[...] + b_ref[...]
    s_ref[...] = xb @ r_ref[...]


def _fused_first(xp, W1, b1, r):
    theta, s = pl.pallas_call(
        _mm1_body,
        grid=(NP // _BLK,),
        in_specs=[
            pl.BlockSpec((_BLK, D), lambda i: (i, _z())),
            pl.BlockSpec((D, H), lambda i: (_z(), _z())),
            pl.BlockSpec((1, H), lambda i: (_z(), _z())),
            pl.BlockSpec((D, 1), lambda i: (_z(), _z())),
        ],
        out_specs=[
            pl.BlockSpec((_BLK, H), lambda i: (i, _z())),
            pl.BlockSpec((_BLK, 1), lambda i: (i, _z())),
        ],
        out_shape=[
            jax.ShapeDtypeStruct((NP, H), jnp.float32),
            jax.ShapeDtypeStruct((NP, 1), jnp.float32),
        ],
    )(xp, W1, b1.reshape(1, H), r.reshape(D, 1))
    return theta, s.reshape(NP)


def _mm2_body(a_ref, dinv_ref, w_ref, b_ref, o_ref):
    dv = dinv_ref[...]
    h = jnp.maximum(a_ref[...] * dv, 0.0)
    o_ref[...] = (h @ w_ref[...] + b_ref[...]) * dv


def _fused_second(acc1, dinv, W2, b2):
    return pl.pallas_call(
        _mm2_body,
        grid=(NP // _BLK,),
        in_specs=[
            pl.BlockSpec((_BLK, H), lambda i: (i, _z())),
            pl.BlockSpec((_BLK, 1), lambda i: (i, _z())),
            pl.BlockSpec((H, H), lambda i: (_z(), _z())),
            pl.BlockSpec((1, H), lambda i: (_z(), _z())),
        ],
        out_specs=pl.BlockSpec((_BLK, H), lambda i: (i, _z())),
        out_shape=jax.ShapeDtypeStruct((NP, H), jnp.float32),
    )(acc1, dinv.reshape(NP, 1), jnp.pad(W2, ((0, 0), (0, H - C))),
      jnp.pad(b2, (0, H - C)).reshape(1, H))


def _make_mp(W):
    mesh = plsc.VectorSubcoreMesh(core_axis_name="c", subcore_axis_name="s",
                                  num_cores=1)

    @functools.partial(
        pl.kernel, mesh=mesh,
        out_type=jax.ShapeDtypeStruct((NP, W), jnp.float32),
        scratch_types=[
            pltpu.VMEM((NCH, CH), jnp.int32),
            pltpu.VMEM((NCH, CH), jnp.int32),
            pltpu.VMEM((CH, W), jnp.float32),
            pltpu.VMEM((CH, W), jnp.float32),
            pltpu.VMEM_SHARED((NP, W), jnp.float32),
            pltpu.SemaphoreType.DMA,
        ],
    )
    def mp(th_hbm, i2_hbm, j2_hbm, o_hbm, idx_i, idx_j, gbuf, rowbuf, accum,
           sem):
        tid = lax.axis_index("s").astype(jnp.int32)
        r0 = tid * jnp.int32(RPT)
        pltpu.sync_copy(i2_hbm.at[tid], idx_i)
        pltpu.sync_copy(j2_hbm.at[tid], idx_j)
        for c in range(NCH):
            rows = pl.ds(r0 + jnp.int32(c * CH), CH)
            pltpu.sync_copy(th_hbm.at[rows], rowbuf)
            pltpu.sync_copy(rowbuf, accum.at[rows])
        plsc.subcore_barrier()
        for c in range(NCH):
            ci = jnp.int32(c)
            pltpu.async_copy(th_hbm.at[idx_i.at[ci]], gbuf, sem).wait()
            pltpu.sync_copy(gbuf, accum.at[idx_j.at[ci]], add=True)
            pltpu.async_copy(th_hbm.at[idx_j.at[ci]], gbuf, sem).wait()
            pltpu.sync_copy(gbuf, accum.at[idx_i.at[ci]], add=True)
        plsc.subcore_barrier()
        for c in range(NCH):
            rows = pl.ds(r0 + jnp.int32(c * CH), CH)
            pltpu.sync_copy(accum.at[rows], rowbuf)
            pltpu.sync_copy(rowbuf, o_hbm.at[rows])

    return mp


_mp128 = _make_mp(H)


def kernel(x, hyperedge_index, W1, b1, W2, b2, r):
    node = hyperedge_index[0].astype(jnp.int32)
    he = hyperedge_index[1].astype(jnp.int32)

    xp = jnp.pad(x, ((0, NP - N), (0, 0)))
    theta_raw, s = _fused_first(xp, W1, b1, r)

    sv = s[node]
    smax = jax.ops.segment_max(sv, he, num_segments=M)
    smin = jax.ops.segment_min(sv, he, num_segments=M)
    big = jnp.int32(N)
    cand_i = jnp.where(sv >= smax[he], node, big)
    i_e = jax.ops.segment_min(cand_i, he, num_segments=M)
    cand_j = jnp.where(sv <= smin[he], node, big)
    j_e = jax.ops.segment_min(cand_j, he, num_segments=M)
    i_e = jnp.clip(i_e, 0, N - 1)
    j_e = jnp.clip(j_e, 0, N - 1)

    deg = jnp.ones((N,), jnp.float32)
    deg = deg.at[i_e].add(1.0).at[j_e].add(1.0)
    dinv = jax.lax.rsqrt(deg)
    dinv = jnp.pad(dinv, (0, NP - N), constant_values=1.0)

    i2 = jnp.pad(i_e, (0, NP - M), constant_values=DUMMY).reshape(NT, NCH, CH)
    j2 = jnp.pad(j_e, (0, NP - M), constant_values=DUMMY).reshape(NT, NCH, CH)

    th1 = theta_raw * dinv[:, None]
    acc1 = _mp128(th1, i2, j2)
    th2 = _fused_second(acc1, dinv, W2, b2)
    acc2 = _mp128(th2, i2, j2)
    out = acc2[:N, :C] * dinv[:N, None]
    return out


# speedup vs baseline: 230.9904x; 20.0739x over previous
import functools
import jax
import jax.numpy as jnp
from jax import lax
from jax.experimental import pallas as pl
from jax.experimental.pallas import tpu as pltpu
from jax.experimental.pallas import tpu_sc as plsc

N = 10000
M = 10000
EI = 320000
D = 128
H = 128
C = 16

NP = 10240
NT = 16
RPT = NP // NT
CH = 128
NCH = RPT // CH
DUMMY = 10100

_BLK = 1280


def _z():
    return jnp.int32(0)


def _mm1_body(x_ref, w_ref, b_ref, r_ref, th_ref, s_ref):
    xb = x_ref[...]
    th_ref[...] = xb @ w_ref[...] + b_ref[...]
    s_ref[...] = xb @ r_ref[...]


def _fused_first(xp, W1, b1, r):
    theta, s = pl.pallas_call(
        _mm1_body,
        grid=(NP // _BLK,),
        in_specs=[
            pl.BlockSpec((_BLK, D), lambda i: (i, _z())),
            pl.BlockSpec((D, H), lambda i: (_z(), _z())),
            pl.BlockSpec((1, H), lambda i: (_z(), _z())),
            pl.BlockSpec((D, 1), lambda i: (_z(), _z())),
        ],
        out_specs=[
            pl.BlockSpec((_BLK, H), lambda i: (i, _z())),
            pl.BlockSpec((_BLK, 1), lambda i: (i, _z())),
        ],
        out_shape=[
            jax.ShapeDtypeStruct((NP, H), jnp.float32),
            jax.ShapeDtypeStruct((NP, 1), jnp.float32),
        ],
    )(xp, W1, b1.reshape(1, H), r.reshape(D, 1))
    return theta, s.reshape(NP)


def _mm2_body(a_ref, dinv_ref, w_ref, b_ref, o_ref):
    dv = dinv_ref[...]
    h = jnp.maximum(a_ref[...] * dv, 0.0)
    o_ref[...] = (h @ w_ref[...] + b_ref[...]) * dv


def _fused_second(acc1, dinv, W2, b2):
    return pl.pallas_call(
        _mm2_body,
        grid=(NP // _BLK,),
        in_specs=[
            pl.BlockSpec((_BLK, H), lambda i: (i, _z())),
            pl.BlockSpec((_BLK, 1), lambda i: (i, _z())),
            pl.BlockSpec((H, H), lambda i: (_z(), _z())),
            pl.BlockSpec((1, H), lambda i: (_z(), _z())),
        ],
        out_specs=pl.BlockSpec((_BLK, H), lambda i: (i, _z())),
        out_shape=jax.ShapeDtypeStruct((NP, H), jnp.float32),
    )(acc1, dinv.reshape(NP, 1), jnp.pad(W2, ((0, 0), (0, H - C))),
      jnp.pad(b2, (0, H - C)).reshape(1, H))


def _make_mp(W):
    mesh = plsc.VectorSubcoreMesh(core_axis_name="c", subcore_axis_name="s",
                                  num_cores=1)

    @functools.partial(
        pl.kernel, mesh=mesh,
        out_type=jax.ShapeDtypeStruct((NP, W), jnp.float32),
        scratch_types=[
            pltpu.VMEM((NCH, CH), jnp.int32),
            pltpu.VMEM((NCH, CH), jnp.int32),
            pltpu.VMEM((CH, W), jnp.float32),
            pltpu.VMEM((CH, W), jnp.float32),
            pltpu.VMEM_SHARED((NP, W), jnp.float32),
            pltpu.SemaphoreType.DMA,
        ],
    )
    def mp(th_hbm, i2_hbm, j2_hbm, o_hbm, idx_i, idx_j, gbuf, rowbuf, accum,
           sem):
        tid = lax.axis_index("s").astype(jnp.int32)
        r0 = tid * jnp.int32(RPT)
        pltpu.sync_copy(i2_hbm.at[tid], idx_i)
        pltpu.sync_copy(j2_hbm.at[tid], idx_j)
        for c in range(NCH):
            rows = pl.ds(r0 + jnp.int32(c * CH), CH)
            pltpu.sync_copy(th_hbm.at[rows], rowbuf)
            pltpu.sync_copy(rowbuf, accum.at[rows])
        plsc.subcore_barrier()
        for c in range(NCH):
            ci = jnp.int32(c)
            pltpu.async_copy(th_hbm.at[idx_i.at[ci]], gbuf, sem).wait()
            pltpu.sync_copy(gbuf, accum.at[idx_j.at[ci]], add=True)
            pltpu.async_copy(th_hbm.at[idx_j.at[ci]], gbuf, sem).wait()
            pltpu.sync_copy(gbuf, accum.at[idx_i.at[ci]], add=True)
        plsc.subcore_barrier()
        for c in range(NCH):
            rows = pl.ds(r0 + jnp.int32(c * CH), CH)
            pltpu.sync_copy(accum.at[rows], rowbuf)
            pltpu.sync_copy(rowbuf, o_hbm.at[rows])

    return mp


_mp128 = _make_mp(H)


EPT = EI // NT
SPT = NP // NT
BIGI = 2147483647


def _iota16():
    return lax.iota(jnp.int32, 16)


_GDN = lax.GatherDimensionNumbers(
    offset_dims=(), collapsed_slice_dims=(0,), start_index_map=(0,))


def _perm(v, idx):
    return lax.gather(v, idx[:, None], _GDN, slice_sizes=(1,),
                      mode=lax.GatherScatterMode.PROMISE_IN_BOUNDS)


def _group_combine(he_v, vals, ops, idents):
    it = _iota16()
    outs = list(vals)
    for d in range(1, 16):
        idxd = (it + jnp.int32(d)) & jnp.int32(15)
        m = he_v == _perm(he_v, idxd)
        for k, op in enumerate(ops):
            outs[k] = op(outs[k], jnp.where(m, _perm(vals[k], idxd),
                                            idents[k]))
    return outs


def _seg_minmax_kernel():
    mesh = plsc.VectorSubcoreMesh(core_axis_name="c", subcore_axis_name="s",
                                  num_cores=1)

    @functools.partial(
        pl.kernel, mesh=mesh,
        compiler_params=pltpu.CompilerParams(needs_layout_passes=False),
        out_type=[jax.ShapeDtypeStruct((NP,), jnp.float32),
                  jax.ShapeDtypeStruct((NP,), jnp.float32)],
        scratch_types=[
            pltpu.VMEM((NP,), jnp.float32),
            pltpu.VMEM((EPT,), jnp.int32),
            pltpu.VMEM((EPT,), jnp.int32),
            pltpu.VMEM((NP,), jnp.float32),
            pltpu.VMEM((NP,), jnp.float32),
            pltpu.VMEM((16, SPT), jnp.float32),
            pltpu.VMEM((SPT,), jnp.float32),
            pltpu.VMEM((SPT,), jnp.float32),
            pltpu.VMEM_SHARED((16, NP), jnp.float32),
        ],
    )
    def k(s_hbm, node_hbm, he_hbm, smax_hbm, smin_hbm,
          s_loc, node_c, he_c, amax, amin, cmb, fmaxc, fminc, pcmb):
        tid = lax.axis_index("s").astype(jnp.int32)
        pltpu.sync_copy(s_hbm, s_loc)
        pltpu.sync_copy(node_hbm.at[pl.ds(tid * jnp.int32(EPT), EPT)], node_c)
        pltpu.sync_copy(he_hbm.at[pl.ds(tid * jnp.int32(EPT), EPT)], he_c)

        ninf = jnp.full((16,), -jnp.inf, jnp.float32)
        pinf = jnp.full((16,), jnp.inf, jnp.float32)

        def init_body(i, _):
            sl = pl.ds(i * jnp.int32(16), 16)
            amax[sl] = ninf
            amin[sl] = pinf
            return 0

        lax.fori_loop(jnp.int32(0), jnp.int32(NP // 16), init_body, 0)

        def body(kk, _):
            sl = pl.ds(kk * jnp.int32(16), 16)
            he_v = he_c[sl]
            nd_v = node_c[sl]
            sv = plsc.load_gather(s_loc, [nd_v])
            gmax, gmin = _group_combine(
                he_v, (sv, sv), (jnp.maximum, jnp.minimum), (ninf, pinf))
            omax = plsc.load_gather(amax, [he_v])
            plsc.store_scatter(amax, [he_v], jnp.maximum(omax, gmax))
            omin = plsc.load_gather(amin, [he_v])
            plsc.store_scatter(amin, [he_v], jnp.minimum(omin, gmin))
            return 0

        lax.fori_loop(jnp.int32(0), jnp.int32(EPT // 16), body, 0)

        pltpu.sync_copy(amax, pcmb.at[tid])
        plsc.subcore_barrier()

        seg0 = tid * jnp.int32(SPT)
        pltpu.sync_copy(pcmb.at[:, pl.ds(seg0, SPT)], cmb)

        def fold_max(j, _):
            sl = pl.ds(j * jnp.int32(16), 16)
            v = cmb[jnp.int32(0), sl]
            for t in range(1, 16):
                v = jnp.maximum(v, cmb[jnp.int32(t), sl])
            fmaxc[sl] = v
            return 0

        lax.fori_loop(jnp.int32(0), jnp.int32(SPT // 16), fold_max, 0)
        pltpu.sync_copy(fmaxc, smax_hbm.at[pl.ds(seg0, SPT)])

        plsc.subcore_barrier()
        pltpu.sync_copy(amin, pcmb.at[tid])
        plsc.subcore_barrier()
        pltpu.sync_copy(pcmb.at[:, pl.ds(seg0, SPT)], cmb)

        def fold_min(j, _):
            sl = pl.ds(j * jnp.int32(16), 16)
            v = cmb[jnp.int32(0), sl]
            for t in range(1, 16):
                v = jnp.minimum(v, cmb[jnp.int32(t), sl])
            fminc[sl] = v
            return 0

        lax.fori_loop(jnp.int32(0), jnp.int32(SPT // 16), fold_min, 0)
        pltpu.sync_copy(fminc, smin_hbm.at[pl.ds(seg0, SPT)])

    return k


def _edge_pick_kernel():
    mesh = plsc.VectorSubcoreMesh(core_axis_name="c", subcore_axis_name="s",
                                  num_cores=1)

    @functools.partial(
        pl.kernel, mesh=mesh,
        compiler_params=pltpu.CompilerParams(needs_layout_passes=False),
        out_type=[jax.ShapeDtypeStruct((NP,), jnp.int32),
                  jax.ShapeDtypeStruct((NP,), jnp.int32),
                  jax.ShapeDtypeStruct((NP,), jnp.float32)],
        scratch_types=[
            pltpu.VMEM((NP,), jnp.float32),
            pltpu.VMEM((EPT,), jnp.int32),
            pltpu.VMEM((EPT,), jnp.int32),
            pltpu.VMEM((NP,), jnp.float32),
            pltpu.VMEM((NP,), jnp.float32),
            pltpu.VMEM((NP,), jnp.int32),
            pltpu.VMEM((NP,), jnp.int32),
            pltpu.VMEM((NP,), jnp.int32),
            pltpu.VMEM((16, SPT), jnp.int32),
            pltpu.VMEM((SPT,), jnp.int32),
            pltpu.VMEM((SPT,), jnp.int32),
            pltpu.VMEM((SPT,), jnp.float32),
            pltpu.VMEM_SHARED((16, NP), jnp.int32),
        ],
    )
    def k(s_hbm, node_hbm, he_hbm, smax_hbm, smin_hbm,
          ie_hbm, je_hbm, deg_hbm,
          s_loc, node_c, he_c, fmax, fmin, ai, aj, degp, cmb,
          icnk, jcnk, degc, pa):
        tid = lax.axis_index("s").astype(jnp.int32)
        pltpu.sync_copy(s_hbm, s_loc)
        pltpu.sync_copy(smax_hbm, fmax)
        pltpu.sync_copy(smin_hbm, fmin)
        pltpu.sync_copy(node_hbm.at[pl.ds(tid * jnp.int32(EPT), EPT)], node_c)
        pltpu.sync_copy(he_hbm.at[pl.ds(tid * jnp.int32(EPT), EPT)], he_c)

        bigv = jnp.full((16,), BIGI, jnp.int32)
        zero = jnp.zeros((16,), jnp.int32)

        def init_body(i, _):
            sl = pl.ds(i * jnp.int32(16), 16)
            ai[sl] = bigv
            aj[sl] = bigv
            degp[sl] = zero
            return 0

        lax.fori_loop(jnp.int32(0), jnp.int32(NP // 16), init_body, 0)

        def body(kk, _):
            sl = pl.ds(kk * jnp.int32(16), 16)
            he_v = he_c[sl]
            nd_v = node_c[sl]
            sv = plsc.load_gather(s_loc, [nd_v])
            mx = plsc.load_gather(fmax, [he_v])
            mn = plsc.load_gather(fmin, [he_v])
            ci = jnp.where(sv >= mx, nd_v, bigv)
            cj = jnp.where(sv <= mn, nd_v, bigv)
            gci, gcj = _group_combine(he_v, (ci, cj),
                                      (jnp.minimum, jnp.minimum),
                                      (bigv, bigv))
            oi = plsc.load_gather(ai, [he_v])
            plsc.store_scatter(ai, [he_v], jnp.minimum(oi, gci))
            oj = plsc.load_gather(aj, [he_v])
            plsc.store_scatter(aj, [he_v], jnp.minimum(oj, gcj))
            return 0

        lax.fori_loop(jnp.int32(0), jnp.int32(EPT // 16), body, 0)

        seg0 = tid * jnp.int32(SPT)
        it = _iota16()
        nmax = jnp.full((16,), N - 1, jnp.int32)
        dummy = jnp.full((16,), DUMMY, jnp.int32)
        mseg = jnp.full((16,), M, jnp.int32)
        one = jnp.ones((16,), jnp.int32)

        for part, dst in ((ai, icnk), (aj, jcnk)):
            pltpu.sync_copy(part, pa.at[tid])
            plsc.subcore_barrier()
            pltpu.sync_copy(pa.at[:, pl.ds(seg0, SPT)], cmb)

            def fold(j, _):
                sl = pl.ds(j * jnp.int32(16), 16)
                v = cmb[jnp.int32(0), sl]
                for t in range(1, 16):
                    v = jnp.minimum(v, cmb[jnp.int32(t), sl])
                v = jnp.clip(v, 0, nmax)
                slot = seg0 + j * jnp.int32(16) + it
                v = jnp.where(slot < mseg, v, dummy)
                dst[sl] = v
                return 0

            lax.fori_loop(jnp.int32(0), jnp.int32(SPT // 16), fold, 0)
            plsc.subcore_barrier()

        pltpu.sync_copy(icnk, ie_hbm.at[pl.ds(seg0, SPT)])
        pltpu.sync_copy(jcnk, je_hbm.at[pl.ds(seg0, SPT)])

        def degscat(j, _):
            sl = pl.ds(j * jnp.int32(16), 16)
            slot = seg0 + j * jnp.int32(16) + it
            val = jnp.where(slot < mseg, one, zero)
            iv = icnk[sl]
            jv = jcnk[sl]
            gi, = _group_combine(iv, (val,), (jnp.add,), (zero,))
            od = plsc.load_gather(degp, [iv])
            plsc.store_scatter(degp, [iv], od + gi)
            gj, = _group_combine(jv, (val,), (jnp.add,), (zero,))
            od2 = plsc.load_gather(degp, [jv])
            plsc.store_scatter(degp, [jv], od2 + gj)
            return 0

        lax.fori_loop(jnp.int32(0), jnp.int32(SPT // 16), degscat, 0)

        pltpu.sync_copy(degp, pa.at[tid])
        plsc.subcore_barrier()

        pltpu.sync_copy(pa.at[:, pl.ds(seg0, SPT)], cmb)

        def folddeg(j, _):
            sl = pl.ds(j * jnp.int32(16), 16)
            v = cmb[jnp.int32(0), sl]
            for t in range(1, 16):
                v = v + cmb[jnp.int32(t), sl]
            degc[sl] = v.astype(jnp.float32) + 1.0
            return 0

        lax.fori_loop(jnp.int32(0), jnp.int32(SPT // 16), folddeg, 0)
        pltpu.sync_copy(degc, deg_hbm.at[pl.ds(seg0, SPT)])

    return k


_seg_minmax = _seg_minmax_kernel()
_edge_pick = _edge_pick_kernel()


def kernel(x, hyperedge_index, W1, b1, W2, b2, r):
    node = hyperedge_index[0].astype(jnp.int32)
    he = hyperedge_index[1].astype(jnp.int32)

    xp = jnp.pad(x, ((0, NP - N), (0, 0)))
    theta_raw, s = _fused_first(xp, W1, b1, r)

    smax, smin = _seg_minmax(s, node, he)
    i_e, j_e, deg = _edge_pick(s, node, he, smax, smin)
    dinv = jax.lax.rsqrt(deg)

    i2 = i_e.reshape(NT, NCH, CH)
    j2 = j_e.reshape(NT, NCH, CH)

    th1 = theta_raw * dinv[:, None]
    acc1 = _mp128(th1, i2, j2)
    th2 = _fused_second(acc1, dinv, W2, b2)
    acc2 = _mp128(th2, i2, j2)
    out = acc2[:N, :C] * dinv[:N, None]
    return out
